# baseline (speedup 1.0000x reference)
import functools

import jax
import jax.numpy as jnp
from jax import lax
from jax.experimental import pallas as pl
from jax.experimental.pallas import tpu as pltpu
from jax.experimental.pallas import tpu_sc as plsc

_QL = 4
_TK = 3
_NC = 2
_NS = 16
_DH = 128


def _sc_streams(B, N, P, D):
    mesh = plsc.VectorSubcoreMesh(core_axis_name="c", subcore_axis_name="s")

    @functools.partial(
        pl.kernel, mesh=mesh,
        out_type=[
            jax.ShapeDtypeStruct((B, _QL, N, D), jnp.float32),
            jax.ShapeDtypeStruct((B, _QL, P, D), jnp.float32),
            jax.ShapeDtypeStruct((B, _TK, N, D), jnp.float32),
        ],
        scratch_types=[
            pltpu.VMEM((N, _DH), jnp.float32),
            pltpu.VMEM((P, _DH), jnp.float32),
            pltpu.SemaphoreType.DMA,
        ],
    )
    def sc_copy(mq, pq, tae, out1, out2, out6, buf, pbuf, sem):
        wid = lax.axis_index("s") * _NC + lax.axis_index("c")
        b = wid // (D // _DH)
        d0 = (wid % (D // _DH)) * _DH

        pltpu.sync_copy(mq.at[b, :, pl.ds(d0, _DH)], buf)
        cs = [pltpu.make_async_copy(buf, out1.at[b, q, :, pl.ds(d0, _DH)], sem)
              for q in range(_QL)]
        for c in cs:
            c.start()
        pltpu.sync_copy(pq.at[b, :, pl.ds(d0, _DH)], pbuf)
        ps = [pltpu.make_async_copy(pbuf, out2.at[b, q, :, pl.ds(d0, _DH)], sem)
              for q in range(_QL)]
        for c in ps:
            c.start()
        for c in cs + ps:
            c.wait()

        for t in range(_TK):
            pltpu.sync_copy(tae.at[b, :, pl.ds(t * D + d0, _DH)], buf)
            pltpu.sync_copy(buf, out6.at[b, t, :, pl.ds(d0, _DH)])

    return sc_copy


def _tc_smalls(ego, ptm, pem, ete, out3, out4, out5, out7, out8):
    ego_v = ego[...]
    for q in range(_QL):
        out3[:, q] = ego_v
    out4[...] = jnp.zeros(out4.shape, jnp.int32)

    ptm_v = ptm[...]
    for t in range(_TK):
        out5[:, t] = ((ptm_v >> (8 * t)) & 1).astype(jnp.int8)

    pem_v = pem[...]
    b0 = (pem_v >> 0) & 1
    b1 = (pem_v >> 8) & 1
    b2 = (pem_v >> 16) & 1
    for t, bt in enumerate((b0, b1, b2)):
        out7[:, t] = bt.astype(jnp.int8)

    all_true = (b0 + b1 + b2) == 3
    ff = jnp.where(b0 == 0, 0, jnp.where(b1 == 0, 1, 2))
    pe0 = ete[:, 0]
    pe1 = ete[:, 1]
    pe2 = ete[:, 2]
    last = ete[:, 3]
    tmp = jnp.where(ff == 0, pe0, jnp.where(ff == 1, pe1, pe2))
    for t, pet in enumerate((pe0, pe1, pe2)):
        val = jnp.where(all_true, last, jnp.where(t < ff, tmp, pet))
        out8[:, t, 0] = val


def kernel(motion_query, plan_query, ego_status_feature, mask,
           temp_anchor_embed_forstate, temp_mask_forstate,
           ego_temp_anchor_embed_forstate, ego_temp_mask_forstate):
    del mask
    B, N, D = motion_query.shape
    P = plan_query.shape[1]

    ptm = jax.lax.bitcast_convert_type(
        temp_mask_forstate.astype(jnp.uint8), jnp.int32)
    pem = jax.lax.bitcast_convert_type(
        ego_temp_mask_forstate.astype(jnp.uint8), jnp.int32)
    ete = ego_temp_anchor_embed_forstate.reshape(B, _QL, D)
    tae = temp_anchor_embed_forstate.reshape(B, N, _QL * D)

    out1, out2, out6 = _sc_streams(B, N, P, D)(motion_query, plan_query, tae)

    vmem = functools.partial(pl.BlockSpec, memory_space=pltpu.VMEM)
    out3, out4, out5, out7, out8 = pl.pallas_call(
        _tc_smalls,
        in_specs=[vmem(), vmem(), vmem(), vmem()],
        out_specs=[vmem(), vmem(), vmem(), vmem(), vmem()],
        out_shape=[
            jax.ShapeDtypeStruct((B, _QL, 1, D), jnp.float32),
            jax.ShapeDtypeStruct((B, _QL), jnp.int32),
            jax.ShapeDtypeStruct((B, _TK, N), jnp.int8),
            jax.ShapeDtypeStruct((B, _TK, 1), jnp.int8),
            jax.ShapeDtypeStruct((B, _TK, 1, D), jnp.float32),
        ],
    )(ego_status_feature, ptm, pem, ete)

    return (out1, out2, out3, out4,
            out5.astype(bool), out6, out7.astype(bool), out8)


# speedup vs baseline: 4.2334x; 4.2334x over previous
import functools

import jax
import jax.numpy as jnp
from jax.experimental import pallas as pl
from jax.experimental.pallas import tpu as pltpu

_QL = 4
_TK = 3
_NCH = 5
_BG = 8


def _tc_body(mqt, pqt, tae, ego, ptm, pem, ete,
             o1, o2, o6, o3, o4, o5, o7, o8):
    b2 = pl.program_id(0)
    nc = pl.program_id(1)

    mqv = mqt[...]
    taev = tae[...]
    cn = mqv.shape[0]
    for i in range(_BG):
        o1[i] = jnp.broadcast_to(mqv[:, i, None, :], (cn, _QL, mqv.shape[2]))
        for t in range(_TK):
            o6[t, :, i, :] = taev[i, :, t, :]

    @pl.when(nc == 0)
    def _plan():
        pqv = pqt[...]
        for i in range(_BG):
            o2[i] = jnp.broadcast_to(
                pqv[:, i, None, :], (pqv.shape[0], _QL, pqv.shape[2]))

    @pl.when(jnp.logical_and(nc == 0, b2 == 0))
    def _smalls():
        ego_v = ego[...]
        for q in range(_QL):
            o3[:, q] = ego_v
        o4[...] = jnp.zeros(o4.shape, jnp.int32)

        ptm_v = ptm[...]
        for t in range(_TK):
            o5[t] = ((ptm_v >> (8 * t)) & 1).astype(jnp.int8)

        pem_v = pem[...]
        b0 = (pem_v >> 0) & 1
        b1 = (pem_v >> 8) & 1
        b2_ = (pem_v >> 16) & 1
        for t, bt in enumerate((b0, b1, b2_)):
            o7[:, t] = bt.astype(jnp.int8)

        all_true = (b0 + b1 + b2_) == 3
        ff = jnp.where(b0 == 0, 0, jnp.where(b1 == 0, 1, 2))
        pe0 = ete[:, 0]
        pe1 = ete[:, 1]
        pe2 = ete[:, 2]
        last = ete[:, 3]
        tmp = jnp.where(ff == 0, pe0, jnp.where(ff == 1, pe1, pe2))
        for t, pet in enumerate((pe0, pe1, pe2)):
            val = jnp.where(all_true, last, jnp.where(t < ff, tmp, pet))
            o8[:, t, 0] = val


def kernel(motion_query, plan_query, ego_status_feature, mask,
           temp_anchor_embed_forstate, temp_mask_forstate,
           ego_temp_anchor_embed_forstate, ego_temp_mask_forstate):
    del mask
    B, N, D = motion_query.shape
    P = plan_query.shape[1]
    CN = N // _NCH
    NB2 = B // _BG
    sq = pl.squeezed

    mqt = jnp.swapaxes(motion_query, 0, 1)
    pqt = jnp.swapaxes(plan_query, 0, 1)

    ptm = jax.lax.bitcast_convert_type(
        temp_mask_forstate.astype(jnp.uint8), jnp.int32)
    pem = jax.lax.bitcast_convert_type(
        ego_temp_mask_forstate.astype(jnp.uint8), jnp.int32)
    ete = ego_temp_anchor_embed_forstate.reshape(B, _QL, D)

    o1, o2, o6, o3, o4, o5, o7, o8 = pl.pallas_call(
        _tc_body,
        grid=(NB2, _NCH),
        in_specs=[
            pl.BlockSpec((CN, _BG, D), lambda b2, nc: (nc, b2, 0)),
            pl.BlockSpec((P, _BG, D), lambda b2, nc: (0, b2, 0)),
            pl.BlockSpec((_BG, CN, _QL, D),
                         lambda b2, nc: (b2, nc, 0, 0)),
            pl.BlockSpec((B, 1, D), lambda b2, nc: (0, 0, 0)),
            pl.BlockSpec((B, N), lambda b2, nc: (0, 0)),
            pl.BlockSpec((B, 1), lambda b2, nc: (0, 0)),
            pl.BlockSpec((B, _QL, D), lambda b2, nc: (0, 0, 0)),
        ],
        out_specs=[
            pl.BlockSpec((_BG, CN, _QL, D),
                         lambda b2, nc: (b2, nc, 0, 0)),
            pl.BlockSpec((_BG, P, _QL, D),
                         lambda b2, nc: (b2, 0, 0, 0)),
            pl.BlockSpec((_TK, CN, _BG, D),
                         lambda b2, nc: (0, nc, b2, 0)),
            pl.BlockSpec((B, _QL, 1, D), lambda b2, nc: (0, 0, 0, 0)),
            pl.BlockSpec((_QL, B), lambda b2, nc: (0, 0)),
            pl.BlockSpec((_TK, B, N), lambda b2, nc: (0, 0, 0)),
            pl.BlockSpec((B, _TK, 1), lambda b2, nc: (0, 0, 0)),
            pl.BlockSpec((B, _TK, 1, D), lambda b2, nc: (0, 0, 0, 0)),
        ],
        out_shape=[
            jax.ShapeDtypeStruct((B, N, _QL, D), jnp.float32),
            jax.ShapeDtypeStruct((B, P, _QL, D), jnp.float32),
            jax.ShapeDtypeStruct((_TK, N, B, D), jnp.float32),
            jax.ShapeDtypeStruct((B, _QL, 1, D), jnp.float32),
            jax.ShapeDtypeStruct((_QL, B), jnp.int32),
            jax.ShapeDtypeStruct((_TK, B, N), jnp.int8),
            jax.ShapeDtypeStruct((B, _TK, 1), jnp.int8),
            jax.ShapeDtypeStruct((B, _TK, 1, D), jnp.float32),
        ],
    )(mqt, pqt, temp_anchor_embed_forstate, ego_status_feature, ptm, pem, ete)

    out1 = jnp.swapaxes(o1, 1, 2)
    out2 = jnp.swapaxes(o2, 1, 2)
    out6 = jnp.transpose(o6, (2, 0, 1, 3))
    out4 = jnp.swapaxes(o4, 0, 1)
    out5 = jnp.swapaxes(o5, 0, 1)
    return (out1, out2, o3, out4,
            out5.astype(bool), out6, o7.astype(bool), o8)
